# baseline (speedup 1.0000x reference)
import functools

import jax
import jax.numpy as jnp
from jax import lax
from jax.experimental import pallas as pl
from jax.experimental.pallas import tpu as pltpu
from jax.experimental.pallas import tpu_sc as plsc

N_NODES = 10000
N_EDGES = 320000
H = 128
D_EDGE = 16
NPAD = 10240

NC = 2
NS = 16
NW = NC * NS
CH = 80
ECH = 128
NCHT = N_EDGES // ECH

_f32 = jnp.float32
_sc_mesh = plsc.VectorSubcoreMesh(core_axis_name="c", subcore_axis_name="s")


def _ln_blk(v, g, b):
    m = jnp.mean(v, axis=-1, keepdims=True)
    s = jnp.var(v, axis=-1, keepdims=True)
    return (v - m) / jnp.sqrt(s + 1e-5) * g + b


def _gather1_body(n_idx, tbl_hbm, idx_hbm, out_hbm, idx_v, buf_v, sem):
    epw = n_idx // NW
    nch = epw // CH
    wid = lax.axis_index("s") * NC + lax.axis_index("c")
    base0 = wid * epw

    def body(g, carry):
        base = base0 + g * CH
        pltpu.sync_copy(idx_hbm.at[pl.ds(base, CH)], idx_v)
        pltpu.async_copy(tbl_hbm.at[idx_v], buf_v, sem).wait()
        pltpu.sync_copy(buf_v, out_hbm.at[pl.ds(base, CH)])
        return carry

    lax.fori_loop(0, nch, body, 0)


def _sc_gather1(tbl, idx):
    n_idx = idx.shape[0]
    k = pl.kernel(
        functools.partial(_gather1_body, n_idx),
        out_type=jax.ShapeDtypeStruct((n_idx, H), _f32),
        mesh=_sc_mesh,
        scratch_types=[
            pltpu.VMEM((CH,), jnp.int32),
            pltpu.VMEM((CH, H), _f32),
            pltpu.SemaphoreType.DMA,
        ],
    )
    return k(tbl, idx)


def _worker_chunks(n_chunks):
    wid = lax.axis_index("s") * NC + lax.axis_index("c")
    nper = n_chunks // NW
    nleft = n_chunks - NW * nper
    extra = jnp.maximum(wid - (NW - nleft), 0)
    cbase = wid * nper + extra
    has_extra = wid >= (NW - nleft)
    return wid, cbase, has_extra, nper


def _gather2_body(n_chunks, coff, h_hbm, row1d, col1d, ghr_hbm, ghc_hbm,
                  idxs_r, idxs_c, bufs_r, bufs_c, gsems_r, gsems_c,
                  wsems_r, wsems_c):
    wid, cbase, has_extra, nper = _worker_chunks(n_chunks)
    npre = (nper + 1) * ECH

    pltpu.sync_copy(row1d.at[pl.ds((coff + cbase) * ECH, npre)], idxs_r)
    pltpu.sync_copy(col1d.at[pl.ds((coff + cbase) * ECH, npre)], idxs_c)

    def gather(k, p):
        cp_r = pltpu.async_copy(
            h_hbm.at[idxs_r.at[pl.ds(k * ECH, ECH)]], bufs_r[p], gsems_r[p])
        cp_c = pltpu.async_copy(
            h_hbm.at[idxs_c.at[pl.ds(k * ECH, ECH)]], bufs_c[p], gsems_c[p])
        return cp_r, cp_c

    def writeback(k, p, cp_r, cp_c):
        r0 = (cbase + k) * ECH
        cp_r.wait()
        cp_c.wait()
        wr = pltpu.async_copy(bufs_r[p], ghr_hbm.at[pl.ds(r0, ECH)],
                              wsems_r[p])
        wc = pltpu.async_copy(bufs_c[p], ghc_hbm.at[pl.ds(r0, ECH)],
                              wsems_c[p])
        return wr, wc

    def body3(t, carry):
        cps = [gather(3 * t + p, p) for p in range(3)]
        wbs = [writeback(3 * t + p, p, *cps[p]) for p in range(3)]
        for wr, wc in wbs:
            wr.wait()
            wc.wait()
        return carry

    assert nper % 3 == 0
    lax.fori_loop(0, nper // 3, body3, 0)

    @pl.when(has_extra)
    def _():
        cpe = gather(nper, 0)
        wr, wc = writeback(nper, 0, *cpe)
        wr.wait()
        wc.wait()


def _sc_gather2(h, row1d, col1d, n_chunks, coff):
    nper = n_chunks // NW
    k = pl.kernel(
        functools.partial(_gather2_body, n_chunks, coff),
        out_type=(jax.ShapeDtypeStruct((n_chunks * ECH, H), _f32),
                  jax.ShapeDtypeStruct((n_chunks * ECH, H), _f32)),
        mesh=_sc_mesh,
        scratch_types=[
            pltpu.VMEM(((nper + 1) * ECH,), jnp.int32),
            pltpu.VMEM(((nper + 1) * ECH,), jnp.int32),
            [pltpu.VMEM((ECH, H), _f32)] * 3,
            [pltpu.VMEM((ECH, H), _f32)] * 3,
            [pltpu.SemaphoreType.DMA] * 3,
            [pltpu.SemaphoreType.DMA] * 3,
            [pltpu.SemaphoreType.DMA] * 3,
            [pltpu.SemaphoreType.DMA] * 3,
        ],
    )
    return k(h, row1d, col1d)


def _scatter_body(n_chunks, coff, out_hbm, row1d, zeros_hbm, agg_hbm,
                  idx_0, idx_1, val_0, val_1, zbuf, acc,
                  sem_0, sem_1, semi_0, semi_1):
    s = lax.axis_index("s")
    c = lax.axis_index("c")
    wid, cbase, has_extra, nper = _worker_chunks(n_chunks)
    rows_per_tile = NPAD // NS
    nzch = rows_per_tile // 64

    def fetch(k, idx_v, val_v, sem_v, sem_i):
        r0 = (cbase + k) * ECH
        ci = pltpu.async_copy(row1d.at[pl.ds((coff + cbase + k) * ECH, ECH)],
                              idx_v, sem_i)
        cv = pltpu.async_copy(out_hbm.at[pl.ds(r0, ECH)], val_v, sem_v)
        return ci, cv

    def commit(idx_v, val_v, ci, cv):
        ci.wait()
        cv.wait()
        pltpu.sync_copy(val_v, acc.at[idx_v], add=True)

    cpa0 = fetch(0, idx_0, val_0, sem_0, semi_0)
    cpb0 = fetch(1, idx_1, val_1, sem_1, semi_1)
    pltpu.sync_copy(zeros_hbm, zbuf)
    for kk in range(nzch):
        pltpu.sync_copy(zbuf, acc.at[pl.ds(s * rows_per_tile + kk * 64, 64)])
    plsc.subcore_barrier()
    commit(idx_0, val_0, *cpa0)
    commit(idx_1, val_1, *cpb0)

    def body(j, carry):
        cpa = fetch(2 * j, idx_0, val_0, sem_0, semi_0)
        cpb = fetch(2 * j + 1, idx_1, val_1, sem_1, semi_1)
        commit(idx_0, val_0, *cpa)
        commit(idx_1, val_1, *cpb)
        return carry

    lax.fori_loop(1, nper // 2, body, 0)

    if nper % 2:
        cpo = fetch(nper - 1, idx_0, val_0, sem_0, semi_0)
        commit(idx_0, val_0, *cpo)

    @pl.when(has_extra)
    def _():
        cpe = fetch(nper, idx_1, val_1, sem_1, semi_1)
        commit(idx_1, val_1, *cpe)

    plsc.subcore_barrier()

    for kk in range(rows_per_tile // ECH):
        r0 = s * rows_per_tile + kk * ECH
        stg = val_0 if kk % 2 == 0 else val_1
        pltpu.sync_copy(acc.at[pl.ds(r0, ECH)], stg)
        pltpu.sync_copy(stg, agg_hbm.at[c, pl.ds(r0, ECH)])


def _sc_scatter(out, row1d, zeros_ch, n_chunks, coff):
    k = pl.kernel(
        functools.partial(_scatter_body, n_chunks, coff),
        out_type=jax.ShapeDtypeStruct((NC, NPAD, H), _f32),
        mesh=_sc_mesh,
        scratch_types=[
            pltpu.VMEM((ECH,), jnp.int32),
            pltpu.VMEM((ECH,), jnp.int32),
            pltpu.VMEM((ECH, H), _f32),
            pltpu.VMEM((ECH, H), _f32),
            pltpu.VMEM((64, H), _f32),
            pltpu.VMEM_SHARED((NPAD, H), _f32),
            pltpu.SemaphoreType.DMA,
            pltpu.SemaphoreType.DMA,
            pltpu.SemaphoreType.DMA,
            pltpu.SemaphoreType.DMA,
        ],
    )
    return k(out, row1d, zeros_ch)


def _full(shape):
    nd = len(shape)
    return pl.BlockSpec(shape, lambda i, _nd=nd: (0,) * _nd)


def _embx_tc(gx, ex_W, ex_b, ex_lng, ex_lnb):
    blk = 1024

    def body(gx_ref, w_ref, b_ref, g_ref, bb_ref, o_ref):
        v = jax.nn.relu(gx_ref[...])
        v = jnp.dot(v, w_ref[...], preferred_element_type=_f32) + b_ref[...]
        o_ref[...] = _ln_blk(v, g_ref[...], bb_ref[...])

    return pl.pallas_call(
        body,
        grid=(NPAD // blk,),
        in_specs=[pl.BlockSpec((blk, H), lambda i: (i, 0)),
                  _full((H, H)), _full((1, H)), _full((1, H)), _full((1, H))],
        out_specs=pl.BlockSpec((blk, H), lambda i: (i, 0)),
        out_shape=jax.ShapeDtypeStruct((NPAD, H), _f32),
    )(gx, ex_W, ex_b, ex_lng, ex_lnb)


def _embe_tc(edge_attr, W1, b1, W2, b2, g, b):
    rows = edge_attr.shape[0]
    blk = 2000

    def body(ea_ref, w1_ref, b1_ref, w2_ref, b2_ref, g_ref, b_ref, o_ref):
        t = jax.nn.relu(
            jnp.dot(ea_ref[...], w1_ref[...], preferred_element_type=_f32)
            + b1_ref[...])
        v = jnp.dot(t.astype(jnp.bfloat16), w2_ref[...],
                    preferred_element_type=_f32) + b2_ref[...]
        o_ref[...] = _ln_blk(v, g_ref[...], b_ref[...]).astype(jnp.bfloat16)

    return pl.pallas_call(
        body,
        grid=(rows // blk,),
        in_specs=[pl.BlockSpec((blk, D_EDGE), lambda i: (i, 0)),
                  _full((D_EDGE, H)), _full((1, H)), _full((H, H)),
                  _full((1, H)), _full((1, H)), _full((1, H))],
        out_specs=pl.BlockSpec((blk, H), lambda i: (i, 0)),
        out_shape=jax.ShapeDtypeStruct((rows, H), jnp.bfloat16),
    )(edge_attr, W1, b1, W2, b2, g, b)


def _mlp_tc(ghr, ghc, e_st, W1r, W1c, W1e, b1, W2, b2, elng, elnb,
            relu_in, last):
    rows = ghr.shape[0]
    blk = 2000
    bf = jnp.bfloat16

    def body(hr_ref, hc_ref, e_ref, w1r_ref, w1c_ref, w1e_ref, b1_ref,
             w2_ref, b2_ref, g_ref, b_ref, out_ref, *maybe_en):
        ein = e_ref[...]
        if relu_in:
            ein = jax.nn.relu(ein)
        hr = hr_ref[...].astype(bf)
        hc = hc_ref[...].astype(bf)
        acc = jnp.dot(hr, w1r_ref[...], preferred_element_type=_f32)
        acc += jnp.dot(hc, w1c_ref[...], preferred_element_type=_f32)
        acc += jnp.dot(ein, w1e_ref[...], preferred_element_type=_f32)
        t = jax.nn.relu(acc + b1_ref[...]).astype(bf)
        o = jnp.dot(t, w2_ref[...], preferred_element_type=_f32) + b2_ref[...]
        out_ref[...] = o
        if not last:
            en = ein.astype(_f32) + _ln_blk(o, g_ref[...], b_ref[...])
            maybe_en[0][...] = en.astype(bf)

    n_out = 1 if last else 2
    out_specs = [pl.BlockSpec((blk, H), lambda i: (i, 0))] * n_out
    out_shape = [jax.ShapeDtypeStruct((rows, H), _f32),
                 jax.ShapeDtypeStruct((rows, H), bf)][:n_out]
    res = pl.pallas_call(
        body,
        grid=(rows // blk,),
        in_specs=[pl.BlockSpec((blk, H), lambda i: (i, 0)),
                  pl.BlockSpec((blk, H), lambda i: (i, 0)),
                  pl.BlockSpec((blk, H), lambda i: (i, 0)),
                  _full((H, 2 * H)), _full((H, 2 * H)), _full((H, 2 * H)),
                  _full((1, 2 * H)), _full((2 * H, H)), _full((1, H)),
                  _full((1, H)), _full((1, H))],
        out_specs=out_specs,
        out_shape=out_shape,
    )(ghr, ghc, e_st, W1r, W1c, W1e, b1, W2, b2, elng, elnb)
    return res if not last else (res[0], None)


def _hupd_tc(hin, aggA, aggB, g, b, final_W, final_b, last):
    def body(h_ref, aa_ref, ab_ref, g_ref, b_ref, *rest):
        if last:
            w_ref, fb_ref, o_ref = rest
        else:
            (o_ref,) = rest
        aa = aa_ref[...]
        ab = ab_ref[...]
        v = h_ref[...] + _ln_blk(aa[0] + aa[1] + ab[0] + ab[1],
                                 g_ref[...], b_ref[...])
        if last:
            o_ref[...] = (jnp.dot(v, w_ref[...], preferred_element_type=_f32)
                          + fb_ref[...])
        else:
            o_ref[...] = jax.nn.relu(v)

    blk = 1024
    in_specs = [pl.BlockSpec((blk, H), lambda i: (i, 0)),
                pl.BlockSpec((NC, blk, H), lambda i: (0, i, 0)),
                pl.BlockSpec((NC, blk, H), lambda i: (0, i, 0)),
                _full((1, H)), _full((1, H))]
    args = [hin, aggA, aggB, g, b]
    if last:
        in_specs += [_full((H, H)), _full((1, H))]
        args += [final_W, final_b]
    return pl.pallas_call(
        body,
        grid=(NPAD // blk,),
        in_specs=in_specs,
        out_specs=pl.BlockSpec((blk, H), lambda i: (i, 0)),
        out_shape=jax.ShapeDtypeStruct((NPAD, H), _f32),
    )(*args)


def kernel(x, edge_index, edge_attr, emb_table, ex_W, ex_b, ex_lng, ex_lnb,
           ea_W1, ea_b1, ea_W2, ea_b2, ea_lng, ea_lnb,
           gc_W1, gc_b1, gc_W2, gc_b2, gc_xlng, gc_xlnb, gc_elng, gc_elnb,
           out_W, out_b):
    r2 = lambda v: v.reshape(1, -1)
    row1d = edge_index[0]
    col1d = edge_index[1]
    xpad = jnp.pad(x, (0, NPAD - N_NODES))
    zeros_ch = jnp.zeros((64, H), _f32)

    bf = jnp.bfloat16

    NCHH = NCHT // 2
    EH = NCHH * ECH

    gx = _sc_gather1(emb_table, xpad)
    h = _embx_tc(gx, ex_W, r2(ex_b), r2(ex_lng), r2(ex_lnb))
    eW2 = ea_W2.astype(bf)
    eA = _embe_tc(edge_attr[:EH], ea_W1, r2(ea_b1), eW2, r2(ea_b2),
                  r2(ea_lng), r2(ea_lnb))
    eB = _embe_tc(edge_attr[EH:], ea_W1, r2(ea_b1), eW2, r2(ea_b2),
                  r2(ea_lng), r2(ea_lnb))

    y = None
    for i in range(4):
        ghrA, ghcA = _sc_gather2(h, row1d, col1d, NCHH, 0)
        ghrB, ghcB = _sc_gather2(h, row1d, col1d, NCHH, NCHH)
        W1 = gc_W1[i].astype(bf)
        wargs = (W1[:H], W1[H:2 * H], W1[2 * H:],
                 r2(gc_b1[i]), gc_W2[i].astype(bf), r2(gc_b2[i]),
                 r2(gc_elng[i]), r2(gc_elnb[i]))
        outA, eA_next = _mlp_tc(ghrA, ghcA, eA, *wargs,
                                relu_in=(i > 0), last=(i == 3))
        outB, eB_next = _mlp_tc(ghrB, ghcB, eB, *wargs,
                                relu_in=(i > 0), last=(i == 3))
        aggA = _sc_scatter(outA, row1d, zeros_ch, NCHH, 0)
        aggB = _sc_scatter(outB, row1d, zeros_ch, NCHH, NCHH)
        if i < 3:
            h = _hupd_tc(h, aggA, aggB, r2(gc_xlng[i]), r2(gc_xlnb[i]),
                         None, None, last=False)
            eA, eB = eA_next, eB_next
        else:
            y = _hupd_tc(h, aggA, aggB, r2(gc_xlng[i]), r2(gc_xlnb[i]),
                         out_W, r2(out_b), last=True)
    return y[:N_NODES]


# speedup vs baseline: 1.0003x; 1.0003x over previous
import functools

import jax
import jax.numpy as jnp
from jax import lax
from jax.experimental import pallas as pl
from jax.experimental.pallas import tpu as pltpu
from jax.experimental.pallas import tpu_sc as plsc

N_NODES = 10000
N_EDGES = 320000
H = 128
D_EDGE = 16
NPAD = 10240

NC = 2
NS = 16
NW = NC * NS
CH = 80
ECH = 128
NCHT = N_EDGES // ECH

_f32 = jnp.float32
_sc_mesh = plsc.VectorSubcoreMesh(core_axis_name="c", subcore_axis_name="s")


def _ln_blk(v, g, b):
    m = jnp.mean(v, axis=-1, keepdims=True)
    s = jnp.var(v, axis=-1, keepdims=True)
    return (v - m) / jnp.sqrt(s + 1e-5) * g + b


def _gather1_body(n_idx, tbl_hbm, idx_hbm, out_hbm, idx_v, buf_v, sem):
    epw = n_idx // NW
    nch = epw // CH
    wid = lax.axis_index("s") * NC + lax.axis_index("c")
    base0 = wid * epw

    def body(g, carry):
        base = base0 + g * CH
        pltpu.sync_copy(idx_hbm.at[pl.ds(base, CH)], idx_v)
        pltpu.async_copy(tbl_hbm.at[idx_v], buf_v, sem).wait()
        pltpu.sync_copy(buf_v, out_hbm.at[pl.ds(base, CH)])
        return carry

    lax.fori_loop(0, nch, body, 0)


def _sc_gather1(tbl, idx):
    n_idx = idx.shape[0]
    k = pl.kernel(
        functools.partial(_gather1_body, n_idx),
        out_type=jax.ShapeDtypeStruct((n_idx, H), _f32),
        mesh=_sc_mesh,
        scratch_types=[
            pltpu.VMEM((CH,), jnp.int32),
            pltpu.VMEM((CH, H), _f32),
            pltpu.SemaphoreType.DMA,
        ],
    )
    return k(tbl, idx)


def _worker_chunks(n_chunks):
    wid = lax.axis_index("s") * NC + lax.axis_index("c")
    nper = n_chunks // NW
    nleft = n_chunks - NW * nper
    extra = jnp.maximum(wid - (NW - nleft), 0)
    cbase = wid * nper + extra
    has_extra = wid >= (NW - nleft)
    return wid, cbase, has_extra, nper


def _gather2_body(n_chunks, coff, h_hbm, row1d, col1d, ghr_hbm, ghc_hbm,
                  idxs_r, idxs_c, bufs_r, bufs_c, gsems_r, gsems_c,
                  wsems_r, wsems_c):
    wid, cbase, has_extra, nper = _worker_chunks(n_chunks)
    npre = (nper + 1) * ECH

    pltpu.sync_copy(row1d.at[pl.ds((coff + cbase) * ECH, npre)], idxs_r)
    pltpu.sync_copy(col1d.at[pl.ds((coff + cbase) * ECH, npre)], idxs_c)

    def gather(k, p):
        cp_r = pltpu.async_copy(
            h_hbm.at[idxs_r.at[pl.ds(k * ECH, ECH)]], bufs_r[p], gsems_r[p])
        cp_c = pltpu.async_copy(
            h_hbm.at[idxs_c.at[pl.ds(k * ECH, ECH)]], bufs_c[p], gsems_c[p])
        return cp_r, cp_c

    def writeback(k, p, cp_r, cp_c):
        r0 = (cbase + k) * ECH
        cp_r.wait()
        cp_c.wait()
        wr = pltpu.async_copy(bufs_r[p], ghr_hbm.at[pl.ds(r0, ECH)],
                              wsems_r[p])
        wc = pltpu.async_copy(bufs_c[p], ghc_hbm.at[pl.ds(r0, ECH)],
                              wsems_c[p])
        return wr, wc

    def wb_wait(k, p):
        r0 = (cbase + k) * ECH
        pltpu.make_async_copy(bufs_r[p], ghr_hbm.at[pl.ds(r0, ECH)],
                              wsems_r[p]).wait()
        pltpu.make_async_copy(bufs_c[p], ghc_hbm.at[pl.ds(r0, ECH)],
                              wsems_c[p]).wait()

    def body3(t, carry):
        @pl.when(t > 0)
        def _():
            for p in range(3):
                wb_wait(3 * (t - 1) + p, p)
        cps = [gather(3 * t + p, p) for p in range(3)]
        for p in range(3):
            writeback(3 * t + p, p, *cps[p])
        return carry

    assert nper % 3 == 0
    lax.fori_loop(0, nper // 3, body3, 0)
    for p in range(3):
        wb_wait(nper - 3 + p, p)

    @pl.when(has_extra)
    def _():
        cpe = gather(nper, 0)
        wr, wc = writeback(nper, 0, *cpe)
        wr.wait()
        wc.wait()


def _sc_gather2(h, row1d, col1d, n_chunks, coff):
    nper = n_chunks // NW
    k = pl.kernel(
        functools.partial(_gather2_body, n_chunks, coff),
        out_type=(jax.ShapeDtypeStruct((n_chunks * ECH, H), _f32),
                  jax.ShapeDtypeStruct((n_chunks * ECH, H), _f32)),
        mesh=_sc_mesh,
        scratch_types=[
            pltpu.VMEM(((nper + 1) * ECH,), jnp.int32),
            pltpu.VMEM(((nper + 1) * ECH,), jnp.int32),
            [pltpu.VMEM((ECH, H), _f32)] * 3,
            [pltpu.VMEM((ECH, H), _f32)] * 3,
            [pltpu.SemaphoreType.DMA] * 3,
            [pltpu.SemaphoreType.DMA] * 3,
            [pltpu.SemaphoreType.DMA] * 3,
            [pltpu.SemaphoreType.DMA] * 3,
        ],
    )
    return k(h, row1d, col1d)


def _scatter_body(n_chunks, coff, out_hbm, row1d, zeros_hbm, agg_hbm,
                  idx_0, idx_1, val_0, val_1, zbuf, acc,
                  sem_0, sem_1, semi_0, semi_1):
    s = lax.axis_index("s")
    c = lax.axis_index("c")
    wid, cbase, has_extra, nper = _worker_chunks(n_chunks)
    rows_per_tile = NPAD // NS
    nzch = rows_per_tile // 64

    def fetch(k, idx_v, val_v, sem_v, sem_i):
        r0 = (cbase + k) * ECH
        ci = pltpu.async_copy(row1d.at[pl.ds((coff + cbase + k) * ECH, ECH)],
                              idx_v, sem_i)
        cv = pltpu.async_copy(out_hbm.at[pl.ds(r0, ECH)], val_v, sem_v)
        return ci, cv

    def commit(idx_v, val_v, ci, cv):
        ci.wait()
        cv.wait()
        pltpu.sync_copy(val_v, acc.at[idx_v], add=True)

    cpa0 = fetch(0, idx_0, val_0, sem_0, semi_0)
    cpb0 = fetch(1, idx_1, val_1, sem_1, semi_1)
    pltpu.sync_copy(zeros_hbm, zbuf)
    for kk in range(nzch):
        pltpu.sync_copy(zbuf, acc.at[pl.ds(s * rows_per_tile + kk * 64, 64)])
    plsc.subcore_barrier()
    commit(idx_0, val_0, *cpa0)
    commit(idx_1, val_1, *cpb0)

    def body(j, carry):
        cpa = fetch(2 * j, idx_0, val_0, sem_0, semi_0)
        cpb = fetch(2 * j + 1, idx_1, val_1, sem_1, semi_1)
        commit(idx_0, val_0, *cpa)
        commit(idx_1, val_1, *cpb)
        return carry

    lax.fori_loop(1, nper // 2, body, 0)

    if nper % 2:
        cpo = fetch(nper - 1, idx_0, val_0, sem_0, semi_0)
        commit(idx_0, val_0, *cpo)

    @pl.when(has_extra)
    def _():
        cpe = fetch(nper, idx_1, val_1, sem_1, semi_1)
        commit(idx_1, val_1, *cpe)

    plsc.subcore_barrier()

    for kk in range(rows_per_tile // ECH):
        r0 = s * rows_per_tile + kk * ECH
        stg = val_0 if kk % 2 == 0 else val_1
        pltpu.sync_copy(acc.at[pl.ds(r0, ECH)], stg)
        pltpu.sync_copy(stg, agg_hbm.at[c, pl.ds(r0, ECH)])


def _sc_scatter(out, row1d, zeros_ch, n_chunks, coff):
    k = pl.kernel(
        functools.partial(_scatter_body, n_chunks, coff),
        out_type=jax.ShapeDtypeStruct((NC, NPAD, H), _f32),
        mesh=_sc_mesh,
        scratch_types=[
            pltpu.VMEM((ECH,), jnp.int32),
            pltpu.VMEM((ECH,), jnp.int32),
            pltpu.VMEM((ECH, H), _f32),
            pltpu.VMEM((ECH, H), _f32),
            pltpu.VMEM((64, H), _f32),
            pltpu.VMEM_SHARED((NPAD, H), _f32),
            pltpu.SemaphoreType.DMA,
            pltpu.SemaphoreType.DMA,
            pltpu.SemaphoreType.DMA,
            pltpu.SemaphoreType.DMA,
        ],
    )
    return k(out, row1d, zeros_ch)


def _full(shape):
    nd = len(shape)
    return pl.BlockSpec(shape, lambda i, _nd=nd: (0,) * _nd)


def _embx_tc(gx, ex_W, ex_b, ex_lng, ex_lnb):
    blk = 1024

    def body(gx_ref, w_ref, b_ref, g_ref, bb_ref, o_ref):
        v = jax.nn.relu(gx_ref[...])
        v = jnp.dot(v, w_ref[...], preferred_element_type=_f32) + b_ref[...]
        o_ref[...] = _ln_blk(v, g_ref[...], bb_ref[...])

    return pl.pallas_call(
        body,
        grid=(NPAD // blk,),
        in_specs=[pl.BlockSpec((blk, H), lambda i: (i, 0)),
                  _full((H, H)), _full((1, H)), _full((1, H)), _full((1, H))],
        out_specs=pl.BlockSpec((blk, H), lambda i: (i, 0)),
        out_shape=jax.ShapeDtypeStruct((NPAD, H), _f32),
    )(gx, ex_W, ex_b, ex_lng, ex_lnb)


def _embe_tc(edge_attr, W1, b1, W2, b2, g, b):
    rows = edge_attr.shape[0]
    blk = 2000

    def body(ea_ref, w1_ref, b1_ref, w2_ref, b2_ref, g_ref, b_ref, o_ref):
        t = jax.nn.relu(
            jnp.dot(ea_ref[...], w1_ref[...], preferred_element_type=_f32)
            + b1_ref[...])
        v = jnp.dot(t.astype(jnp.bfloat16), w2_ref[...],
                    preferred_element_type=_f32) + b2_ref[...]
        o_ref[...] = _ln_blk(v, g_ref[...], b_ref[...]).astype(jnp.bfloat16)

    return pl.pallas_call(
        body,
        grid=(rows // blk,),
        in_specs=[pl.BlockSpec((blk, D_EDGE), lambda i: (i, 0)),
                  _full((D_EDGE, H)), _full((1, H)), _full((H, H)),
                  _full((1, H)), _full((1, H)), _full((1, H))],
        out_specs=pl.BlockSpec((blk, H), lambda i: (i, 0)),
        out_shape=jax.ShapeDtypeStruct((rows, H), jnp.bfloat16),
    )(edge_attr, W1, b1, W2, b2, g, b)


def _mlp_tc(ghr, ghc, e_st, W1r, W1c, W1e, b1, W2, b2, elng, elnb,
            relu_in, last):
    rows = ghr.shape[0]
    blk = 2000
    bf = jnp.bfloat16

    def body(hr_ref, hc_ref, e_ref, w1r_ref, w1c_ref, w1e_ref, b1_ref,
             w2_ref, b2_ref, g_ref, b_ref, out_ref, *maybe_en):
        ein = e_ref[...]
        if relu_in:
            ein = jax.nn.relu(ein)
        hr = hr_ref[...].astype(bf)
        hc = hc_ref[...].astype(bf)
        acc = jnp.dot(hr, w1r_ref[...], preferred_element_type=_f32)
        acc += jnp.dot(hc, w1c_ref[...], preferred_element_type=_f32)
        acc += jnp.dot(ein, w1e_ref[...], preferred_element_type=_f32)
        t = jax.nn.relu(acc + b1_ref[...]).astype(bf)
        o = jnp.dot(t, w2_ref[...], preferred_element_type=_f32) + b2_ref[...]
        out_ref[...] = o
        if not last:
            en = ein.astype(_f32) + _ln_blk(o, g_ref[...], b_ref[...])
            maybe_en[0][...] = en.astype(bf)

    n_out = 1 if last else 2
    out_specs = [pl.BlockSpec((blk, H), lambda i: (i, 0))] * n_out
    out_shape = [jax.ShapeDtypeStruct((rows, H), _f32),
                 jax.ShapeDtypeStruct((rows, H), bf)][:n_out]
    res = pl.pallas_call(
        body,
        grid=(rows // blk,),
        in_specs=[pl.BlockSpec((blk, H), lambda i: (i, 0)),
                  pl.BlockSpec((blk, H), lambda i: (i, 0)),
                  pl.BlockSpec((blk, H), lambda i: (i, 0)),
                  _full((H, 2 * H)), _full((H, 2 * H)), _full((H, 2 * H)),
                  _full((1, 2 * H)), _full((2 * H, H)), _full((1, H)),
                  _full((1, H)), _full((1, H))],
        out_specs=out_specs,
        out_shape=out_shape,
    )(ghr, ghc, e_st, W1r, W1c, W1e, b1, W2, b2, elng, elnb)
    return res if not last else (res[0], None)


def _hupd_tc(hin, aggA, aggB, g, b, final_W, final_b, last):
    def body(h_ref, aa_ref, ab_ref, g_ref, b_ref, *rest):
        if last:
            w_ref, fb_ref, o_ref = rest
        else:
            (o_ref,) = rest
        aa = aa_ref[...]
        ab = ab_ref[...]
        v = h_ref[...] + _ln_blk(aa[0] + aa[1] + ab[0] + ab[1],
                                 g_ref[...], b_ref[...])
        if last:
            o_ref[...] = (jnp.dot(v, w_ref[...], preferred_element_type=_f32)
                          + fb_ref[...])
        else:
            o_ref[...] = jax.nn.relu(v)

    blk = 1024
    in_specs = [pl.BlockSpec((blk, H), lambda i: (i, 0)),
                pl.BlockSpec((NC, blk, H), lambda i: (0, i, 0)),
                pl.BlockSpec((NC, blk, H), lambda i: (0, i, 0)),
                _full((1, H)), _full((1, H))]
    args = [hin, aggA, aggB, g, b]
    if last:
        in_specs += [_full((H, H)), _full((1, H))]
        args += [final_W, final_b]
    return pl.pallas_call(
        body,
        grid=(NPAD // blk,),
        in_specs=in_specs,
        out_specs=pl.BlockSpec((blk, H), lambda i: (i, 0)),
        out_shape=jax.ShapeDtypeStruct((NPAD, H), _f32),
    )(*args)


def kernel(x, edge_index, edge_attr, emb_table, ex_W, ex_b, ex_lng, ex_lnb,
           ea_W1, ea_b1, ea_W2, ea_b2, ea_lng, ea_lnb,
           gc_W1, gc_b1, gc_W2, gc_b2, gc_xlng, gc_xlnb, gc_elng, gc_elnb,
           out_W, out_b):
    r2 = lambda v: v.reshape(1, -1)
    row1d = edge_index[0]
    col1d = edge_index[1]
    xpad = jnp.pad(x, (0, NPAD - N_NODES))
    zeros_ch = jnp.zeros((64, H), _f32)

    bf = jnp.bfloat16

    NCHH = NCHT // 2
    EH = NCHH * ECH

    gx = _sc_gather1(emb_table, xpad)
    h = _embx_tc(gx, ex_W, r2(ex_b), r2(ex_lng), r2(ex_lnb))
    eW2 = ea_W2.astype(bf)
    eA = _embe_tc(edge_attr[:EH], ea_W1, r2(ea_b1), eW2, r2(ea_b2),
                  r2(ea_lng), r2(ea_lnb))
    eB = _embe_tc(edge_attr[EH:], ea_W1, r2(ea_b1), eW2, r2(ea_b2),
                  r2(ea_lng), r2(ea_lnb))

    y = None
    for i in range(4):
        ghrA, ghcA = _sc_gather2(h, row1d, col1d, NCHH, 0)
        ghrB, ghcB = _sc_gather2(h, row1d, col1d, NCHH, NCHH)
        W1 = gc_W1[i].astype(bf)
        wargs = (W1[:H], W1[H:2 * H], W1[2 * H:],
                 r2(gc_b1[i]), gc_W2[i].astype(bf), r2(gc_b2[i]),
                 r2(gc_elng[i]), r2(gc_elnb[i]))
        outA, eA_next = _mlp_tc(ghrA, ghcA, eA, *wargs,
                                relu_in=(i > 0), last=(i == 3))
        outB, eB_next = _mlp_tc(ghrB, ghcB, eB, *wargs,
                                relu_in=(i > 0), last=(i == 3))
        aggA = _sc_scatter(outA, row1d, zeros_ch, NCHH, 0)
        aggB = _sc_scatter(outB, row1d, zeros_ch, NCHH, NCHH)
        if i < 3:
            h = _hupd_tc(h, aggA, aggB, r2(gc_xlng[i]), r2(gc_xlnb[i]),
                         None, None, last=False)
            eA, eB = eA_next, eB_next
        else:
            y = _hupd_tc(h, aggA, aggB, r2(gc_xlng[i]), r2(gc_xlnb[i]),
                         out_W, r2(out_b), last=True)
    return y[:N_NODES]
